# baseline (speedup 1.0000x reference)
import functools

import numpy as np
import jax
import jax.numpy as jnp
from jax import lax
from jax.experimental import pallas as pl
from jax.experimental.pallas import tpu as pltpu
from jax.experimental.pallas import tpu_sc as plsc

_MAX_LEN = 4096


def _pe_table(d_model):
    pos = np.arange(0, _MAX_LEN, dtype=np.float32)[:, None]
    mul = np.exp(
        np.arange(0, d_model, 2, dtype=np.float32) * -(np.log(10000.0) / d_model)
    )
    pe = np.zeros((_MAX_LEN, d_model), dtype=np.float32)
    pe[:, 0::2] = np.sin(pos * mul)
    pe[:, 1::2] = np.cos(pos * mul)
    return jnp.asarray(pe)


def kernel(tokens, embed_table):
    B, S = tokens.shape
    V, D = embed_table.shape
    N = B * S
    flat_tok = tokens.reshape(N).astype(jnp.int32)
    pe = _pe_table(D)[:S]

    info = plsc.get_sparse_core_info()
    NC, NS = info.num_cores, info.num_subcores
    NW = NC * NS
    s_per_w = S // NW
    K = 16
    NP = s_per_w // K
    LANES = D // 16

    mesh = plsc.VectorSubcoreMesh(core_axis_name="c", subcore_axis_name="s")

    @functools.partial(
        pl.kernel,
        mesh=mesh,
        out_type=jax.ShapeDtypeStruct((N, D), jnp.float32),
        scratch_types=[
            pltpu.VMEM((B * s_per_w,), jnp.int32),
            pltpu.VMEM((B * K, D), jnp.float32),
            pltpu.VMEM((2 * K, D), jnp.float32),
            pltpu.SemaphoreType.DMA,
            pltpu.SemaphoreType.DMA,
            pltpu.SemaphoreType.DMA,
            pltpu.SemaphoreType.DMA,
            pltpu.SemaphoreType.DMA,
            pltpu.SemaphoreType.DMA,
            pltpu.SemaphoreType.DMA,
            pltpu.SemaphoreType.DMA,
            pltpu.SemaphoreType.DMA,
        ],
    )
    def emb_kernel(
        tok_hbm,
        table_hbm,
        pe_hbm,
        out_hbm,
        idx_v,
        rows_v,
        pe_v,
        sg0,
        sg1,
        sg2,
        sg3,
        sw0,
        sw1,
        sw2,
        sw3,
        spe,
    ):
        wid = lax.axis_index("s") * NC + lax.axis_index("c")
        s_base = wid * s_per_w
        sg = (sg0, sg1, sg2, sg3)
        sw = (sw0, sw1, sw2, sw3)

        def wait_write(q):
            pltpu.make_async_copy(
                rows_v.at[pl.ds(q * K, K)], out_hbm.at[pl.ds(0, K)], sw[q]
            ).wait()

        def wait_gather(q):
            pltpu.make_async_copy(
                table_hbm.at[pl.ds(0, K)], rows_v.at[pl.ds(q * K, K)], sg[q]
            ).wait()

        def issue_gather(p, b, q):
            pltpu.async_copy(
                table_hbm.at[idx_v.at[pl.ds(b * s_per_w + p * K, K)]],
                rows_v.at[pl.ds(q * K, K)],
                sg[q],
            )

        for b in range(B):
            pltpu.sync_copy(
                tok_hbm.at[pl.ds(b * S + s_base, s_per_w)],
                idx_v.at[pl.ds(b * s_per_w, s_per_w)],
            )

        pltpu.async_copy(pe_hbm.at[pl.ds(s_base, K)], pe_v.at[pl.ds(0, K)], spe)
        issue_gather(0, 0, 0)
        issue_gather(0, 1, 1)

        def outer(p, _):
            pe_off = lax.rem(p, 2) * K
            pltpu.make_async_copy(
                pe_hbm.at[pl.ds(0, K)], pe_v.at[pl.ds(0, K)], spe
            ).wait()

            @pl.when(p + 1 < NP)
            def _():
                nxt = lax.rem(p + 1, 2) * K
                pltpu.async_copy(
                    pe_hbm.at[pl.ds(s_base + (p + 1) * K, K)],
                    pe_v.at[pl.ds(nxt, K)],
                    spe,
                )

            for b in range(B):
                bn = (b + 2) % B
                if b < 2:

                    @pl.when(p >= 1)
                    def _():
                        wait_write(bn)

                    issue_gather(p, bn, bn)
                else:

                    @pl.when(p + 1 < NP)
                    def _():
                        wait_write(bn)
                        issue_gather(p + 1, bn, bn)

                wait_gather(b)

                def add_row(r, _):
                    ro = b * K + r
                    po = pe_off + r
                    for j in range(LANES):
                        sl = pl.ds(j * 16, 16)
                        rows_v[ro, sl] = rows_v[ro, sl] + pe_v[po, sl]
                    return 0

                lax.fori_loop(0, K, add_row, 0)
                pltpu.async_copy(
                    rows_v.at[pl.ds(b * K, K)],
                    out_hbm.at[pl.ds(b * S + s_base + p * K, K)],
                    sw[b],
                )
            return 0

        lax.fori_loop(0, NP, outer, 0)
        for b in range(B):
            wait_write(b)

    out = emb_kernel(flat_tok, embed_table, pe)
    return out.reshape(B, S, D)


# speedup vs baseline: 1.2580x; 1.2580x over previous
import functools

import numpy as np
import jax
import jax.numpy as jnp
from jax import lax
from jax.experimental import pallas as pl
from jax.experimental.pallas import tpu as pltpu
from jax.experimental.pallas import tpu_sc as plsc

_MAX_LEN = 4096


def _pe_table(d_model):
    pos = np.arange(0, _MAX_LEN, dtype=np.float32)[:, None]
    mul = np.exp(
        np.arange(0, d_model, 2, dtype=np.float32) * -(np.log(10000.0) / d_model)
    )
    pe = np.zeros((_MAX_LEN, d_model), dtype=np.float32)
    pe[:, 0::2] = np.sin(pos * mul)
    pe[:, 1::2] = np.cos(pos * mul)
    return jnp.asarray(pe)


def kernel(tokens, embed_table):
    B, S = tokens.shape
    V, D = embed_table.shape
    N = B * S
    flat_tok = tokens.reshape(N).astype(jnp.int32)
    pe = _pe_table(D)[:S]

    info = plsc.get_sparse_core_info()
    NC, NS = info.num_cores, info.num_subcores
    NW = NC * NS
    s_per_w = S // NW
    K = 16
    NP = s_per_w // K
    LANES = D // 16

    mesh = plsc.VectorSubcoreMesh(core_axis_name="c", subcore_axis_name="s")

    @functools.partial(
        pl.kernel,
        mesh=mesh,
        out_type=jax.ShapeDtypeStruct((N, D), jnp.float32),
        scratch_types=[
            pltpu.VMEM((B * s_per_w,), jnp.int32),
            pltpu.VMEM((B * K, D), jnp.float32),
            pltpu.VMEM((2 * K, D), jnp.float32),
            pltpu.SemaphoreType.DMA,
            pltpu.SemaphoreType.DMA,
            pltpu.SemaphoreType.DMA,
            pltpu.SemaphoreType.DMA,
            pltpu.SemaphoreType.DMA,
            pltpu.SemaphoreType.DMA,
            pltpu.SemaphoreType.DMA,
            pltpu.SemaphoreType.DMA,
            pltpu.SemaphoreType.DMA,
        ],
    )
    def emb_kernel(
        tok_hbm,
        table_hbm,
        pe_hbm,
        out_hbm,
        idx_v,
        rows_v,
        pe_v,
        sg0,
        sg1,
        sg2,
        sg3,
        sw0,
        sw1,
        sw2,
        sw3,
        spe,
    ):
        wid = lax.axis_index("s") * NC + lax.axis_index("c")
        s_base = wid * s_per_w
        sg = (sg0, sg1, sg2, sg3)
        sw = (sw0, sw1, sw2, sw3)

        def wait_write(q):
            pltpu.make_async_copy(
                rows_v.at[pl.ds(q * K, K)], out_hbm.at[pl.ds(0, K)], sw[q]
            ).wait()

        def wait_gather(q):
            pltpu.make_async_copy(
                table_hbm.at[pl.ds(0, K)], rows_v.at[pl.ds(q * K, K)], sg[q]
            ).wait()

        def issue_gather(p, b, q):
            pltpu.async_copy(
                table_hbm.at[idx_v.at[pl.ds(b * s_per_w + p * K, K)]],
                rows_v.at[pl.ds(q * K, K)],
                sg[q],
            )

        for b in range(B):
            pltpu.sync_copy(
                tok_hbm.at[pl.ds(b * S + s_base, s_per_w)],
                idx_v.at[pl.ds(b * s_per_w, s_per_w)],
            )

        pltpu.async_copy(pe_hbm.at[pl.ds(s_base, K)], pe_v.at[pl.ds(0, K)], spe)
        issue_gather(0, 0, 0)
        issue_gather(0, 1, 1)

        def outer(p, _):
            pe_off = lax.rem(p, 2) * K
            pltpu.make_async_copy(
                pe_hbm.at[pl.ds(0, K)], pe_v.at[pl.ds(0, K)], spe
            ).wait()

            @pl.when(p + 1 < NP)
            def _():
                nxt = lax.rem(p + 1, 2) * K
                pltpu.async_copy(
                    pe_hbm.at[pl.ds(s_base + (p + 1) * K, K)],
                    pe_v.at[pl.ds(nxt, K)],
                    spe,
                )

            for b in range(B):
                bn = (b + 2) % B
                if b < 2:

                    @pl.when(p >= 1)
                    def _():
                        wait_write(bn)

                    issue_gather(p, bn, bn)
                else:

                    @pl.when(p + 1 < NP)
                    def _():
                        wait_write(bn)
                        issue_gather(p + 1, bn, bn)

                wait_gather(b)

                def add_row(r, _):
                    ro = b * K + r
                    po = pe_off + r
                    for j in range(LANES):
                        sl = pl.ds(j * 16, 16)
                        plsc.addupdate(rows_v.at[ro, sl], pe_v[po, sl])
                    return 0

                lax.fori_loop(0, K, add_row, 0)
                pltpu.async_copy(
                    rows_v.at[pl.ds(b * K, K)],
                    out_hbm.at[pl.ds(b * S + s_base + p * K, K)],
                    sw[b],
                )
            return 0

        lax.fori_loop(0, NP, outer, 0)
        for b in range(B):
            wait_write(b)

    out = emb_kernel(flat_tok, embed_table, pe)
    return out.reshape(B, S, D)


# speedup vs baseline: 2.0094x; 1.5974x over previous
import functools

import numpy as np
import jax
import jax.numpy as jnp
from jax import lax
from jax.experimental import pallas as pl
from jax.experimental.pallas import tpu as pltpu
from jax.experimental.pallas import tpu_sc as plsc

_MAX_LEN = 4096


def _pe_table(d_model):
    pos = np.arange(0, _MAX_LEN, dtype=np.float32)[:, None]
    mul = np.exp(
        np.arange(0, d_model, 2, dtype=np.float32) * -(np.log(10000.0) / d_model)
    )
    pe = np.zeros((_MAX_LEN, d_model), dtype=np.float32)
    pe[:, 0::2] = np.sin(pos * mul)
    pe[:, 1::2] = np.cos(pos * mul)
    return jnp.asarray(pe)


def kernel(tokens, embed_table):
    B, S = tokens.shape
    V, D = embed_table.shape
    N = B * S
    flat_tok = tokens.reshape(N).astype(jnp.int32)
    pe = _pe_table(D)[:S]

    info = plsc.get_sparse_core_info()
    NC, NS = info.num_cores, info.num_subcores
    NW = NC * NS
    s_per_w = S // NW
    K = 16
    NP = s_per_w // K
    LANES = D // 16

    mesh = plsc.VectorSubcoreMesh(core_axis_name="c", subcore_axis_name="s")

    @functools.partial(
        pl.kernel,
        mesh=mesh,
        out_type=jax.ShapeDtypeStruct((N, D), jnp.float32),
        scratch_types=[
            pltpu.VMEM((B * s_per_w,), jnp.int32),
            pltpu.VMEM((B * K, D), jnp.float32),
            pltpu.VMEM((2 * K, D), jnp.float32),
            pltpu.SemaphoreType.DMA,
            pltpu.SemaphoreType.DMA,
            pltpu.SemaphoreType.DMA,
            pltpu.SemaphoreType.DMA,
            pltpu.SemaphoreType.DMA,
            pltpu.SemaphoreType.DMA,
            pltpu.SemaphoreType.DMA,
            pltpu.SemaphoreType.DMA,
            pltpu.SemaphoreType.DMA,
        ],
    )
    def emb_kernel(
        tok_hbm,
        table_hbm,
        pe_hbm,
        out_hbm,
        idx_v,
        rows_v,
        pe_v,
        sg0,
        sg1,
        sg2,
        sg3,
        sw0,
        sw1,
        sw2,
        sw3,
        spe,
    ):
        wid = lax.axis_index("s") * NC + lax.axis_index("c")
        s_base = wid * s_per_w
        sg = (sg0, sg1, sg2, sg3)
        sw = (sw0, sw1, sw2, sw3)

        def wait_write(q):
            pltpu.make_async_copy(
                rows_v.at[pl.ds(q * K, K)], out_hbm.at[pl.ds(0, K)], sw[q]
            ).wait()

        def wait_gather(q):
            pltpu.make_async_copy(
                table_hbm.at[pl.ds(0, K)], rows_v.at[pl.ds(q * K, K)], sg[q]
            ).wait()

        def issue_gather(p, b, q):
            pltpu.async_copy(
                table_hbm.at[idx_v.at[pl.ds(b * s_per_w + p * K, K)]],
                rows_v.at[pl.ds(q * K, K)],
                sg[q],
            )

        for b in range(B):
            pltpu.sync_copy(
                tok_hbm.at[pl.ds(b * S + s_base, s_per_w)],
                idx_v.at[pl.ds(b * s_per_w, s_per_w)],
            )

        pltpu.async_copy(pe_hbm.at[pl.ds(s_base, K)], pe_v.at[pl.ds(0, K)], spe)
        issue_gather(0, 0, 0)
        issue_gather(0, 1, 1)

        def outer(p, _):
            pe_off = lax.rem(p, 2) * K
            pltpu.make_async_copy(
                pe_hbm.at[pl.ds(0, K)], pe_v.at[pl.ds(0, K)], spe
            ).wait()

            @pl.when(p + 1 < NP)
            def _():
                nxt = lax.rem(p + 1, 2) * K
                pltpu.async_copy(
                    pe_hbm.at[pl.ds(s_base + (p + 1) * K, K)],
                    pe_v.at[pl.ds(nxt, K)],
                    spe,
                )

            for b in range(B):
                bn = (b + 2) % B
                if b < 2:

                    @pl.when(p >= 1)
                    def _():
                        wait_write(bn)

                    issue_gather(p, bn, bn)
                else:

                    @pl.when(p + 1 < NP)
                    def _():
                        wait_write(bn)
                        issue_gather(p + 1, bn, bn)

                wait_gather(b)

                @plsc.parallel_loop(0, K, 1)
                def add_row(r):
                    ro = b * K + r
                    po = pe_off + r
                    for j in range(LANES):
                        sl = pl.ds(j * 16, 16)
                        plsc.addupdate(rows_v.at[ro, sl], pe_v[po, sl])
                pltpu.async_copy(
                    rows_v.at[pl.ds(b * K, K)],
                    out_hbm.at[pl.ds(b * S + s_base + p * K, K)],
                    sw[b],
                )
            return 0

        lax.fori_loop(0, NP, outer, 0)
        for b in range(B):
            wait_write(b)

    out = emb_kernel(flat_tok, embed_table, pe)
    return out.reshape(B, S, D)
